# Initial kernel scaffold; baseline (speedup 1.0000x reference)
import functools

import jax
import jax.numpy as jnp
from jax import lax
from jax.experimental import pallas as pl
from jax.experimental.pallas import tpu as pltpu
from jax.experimental.pallas import tpu_sc as plsc

NUM_CORES = 2
NUM_SUBCORES = 16
NW = NUM_CORES * NUM_SUBCORES

ROWS = 128
SLAB = 50


def _build_kernel(n_rows, d_model, n_blocks):
    mesh = plsc.VectorSubcoreMesh(core_axis_name="c", subcore_axis_name="s")
    n_slabs = n_blocks // SLAB

    @functools.partial(
        pl.kernel,
        out_type=jax.ShapeDtypeStruct((NW, n_blocks, ROWS, d_model), jnp.float32),
        mesh=mesh,
        scratch_types=[
            pltpu.VMEM_SHARED((n_rows, d_model), jnp.float32),
            pltpu.VMEM((SLAB, ROWS), jnp.int32),
            pltpu.VMEM((ROWS, d_model), jnp.float32),
            pltpu.VMEM((ROWS, d_model), jnp.float32),
            pltpu.SemaphoreType.DMA,
            pltpu.SemaphoreType.DMA,
            pltpu.SemaphoreType.DMA,
            pltpu.SemaphoreType.DMA,
        ],
    )
    def gather_kernel(idx_hbm, pe_hbm, out_hbm, table_sp, idx_v, rows0, rows1,
                      gsem0, gsem1, wsem0, wsem1):
        c = lax.axis_index("c")
        s = lax.axis_index("s")
        wid = c * NUM_SUBCORES + s

        @pl.when(s == 0)
        def _():
            pltpu.sync_copy(pe_hbm, table_sp)
        plsc.subcore_barrier()

        def slab_body(si, _):
            pltpu.sync_copy(idx_hbm.at[wid, pl.ds(si * SLAB, SLAB)], idx_v)

            def pair_body(p, _):
                j0 = si * SLAB + 2 * p
                j1 = si * SLAB + 2 * p + 1
                pltpu.async_copy(table_sp.at[idx_v.at[2 * p]], rows0, gsem0).wait()
                w0 = pltpu.async_copy(rows0, out_hbm.at[wid, j0], wsem0)

                @pl.when(j1 > 1)
                def _():
                    pltpu.make_async_copy(
                        rows1, out_hbm.at[wid, j1], wsem1
                    ).wait()

                pltpu.async_copy(table_sp.at[idx_v.at[2 * p + 1]], rows1, gsem1).wait()
                pltpu.async_copy(rows1, out_hbm.at[wid, j1], wsem1)
                w0.wait()
                return 0

            lax.fori_loop(0, SLAB // 2, pair_body, 0)
            return 0

        lax.fori_loop(0, n_slabs, slab_body, 0)
        pltpu.make_async_copy(
            rows1, out_hbm.at[wid, n_blocks - 1], wsem1
        ).wait()

    return gather_kernel


def kernel(doy, pe):
    batch, hist = doy.shape
    n_rows, d_model = pe.shape
    total = batch * hist
    assert total % (NW * ROWS) == 0
    n_blocks = total // (NW * ROWS)
    assert n_blocks % SLAB == 0

    idx = doy.reshape(NW, n_blocks, ROWS).astype(jnp.int32)
    out = _build_kernel(n_rows, d_model, n_blocks)(idx, pe)
    return out.reshape(batch, hist, d_model)


# speedup vs baseline: 18.1565x; 18.1565x over previous
import functools

import jax
import jax.numpy as jnp
from jax import lax
from jax.experimental import pallas as pl
from jax.experimental.pallas import tpu as pltpu
from jax.experimental.pallas import tpu_sc as plsc

NUM_CORES = 2
NUM_SUBCORES = 16
NW = NUM_CORES * NUM_SUBCORES

ROWS = 128
SLAB = 80


def _build_kernel(n_rows, d_model, n_blocks):
    mesh = plsc.VectorSubcoreMesh(core_axis_name="c", subcore_axis_name="s")
    n_slabs = n_blocks // SLAB

    @functools.partial(
        pl.kernel,
        out_type=jax.ShapeDtypeStruct((NW, n_blocks, ROWS, d_model), jnp.float32),
        mesh=mesh,
        scratch_types=[
            pltpu.VMEM_SHARED((n_rows, d_model), jnp.float32),
            pltpu.VMEM((SLAB, ROWS), jnp.int32),
            pltpu.VMEM((ROWS, d_model), jnp.float32),
            pltpu.VMEM((ROWS, d_model), jnp.float32),
            pltpu.SemaphoreType.DMA,
            pltpu.SemaphoreType.DMA,
            pltpu.SemaphoreType.DMA,
            pltpu.SemaphoreType.DMA,
        ],
    )
    def gather_kernel(idx_hbm, pe_hbm, out_hbm, table_sp, idx_v, rows0, rows1,
                      gsem0, gsem1, wsem0, wsem1):
        c = lax.axis_index("c")
        s = lax.axis_index("s")
        wid = c * NUM_SUBCORES + s

        @pl.when(s == 0)
        def _():
            pltpu.sync_copy(pe_hbm, table_sp)
        plsc.subcore_barrier()

        def slab_body(si, _):
            pltpu.sync_copy(idx_hbm.at[wid, pl.ds(si * SLAB, SLAB)], idx_v)

            def pair_body(p, _):
                j0 = si * SLAB + 2 * p
                j1 = si * SLAB + 2 * p + 1
                pltpu.async_copy(table_sp.at[idx_v.at[2 * p]], rows0, gsem0).wait()
                w0 = pltpu.async_copy(rows0, out_hbm.at[wid, j0], wsem0)

                @pl.when(j1 > 1)
                def _():
                    pltpu.make_async_copy(
                        rows1, out_hbm.at[wid, j1], wsem1
                    ).wait()

                pltpu.async_copy(table_sp.at[idx_v.at[2 * p + 1]], rows1, gsem1).wait()
                pltpu.async_copy(rows1, out_hbm.at[wid, j1], wsem1)
                w0.wait()
                return 0

            lax.fori_loop(0, SLAB // 2, pair_body, 0)
            return 0

        lax.fori_loop(0, n_slabs, slab_body, 0)
        pltpu.make_async_copy(
            rows1, out_hbm.at[wid, n_blocks - 1], wsem1
        ).wait()

    return gather_kernel


def kernel(doy, pe):
    batch, hist = doy.shape
    n_rows, d_model = pe.shape
    total = batch * hist
    assert total % (NW * ROWS) == 0
    n_blocks = total // (NW * ROWS)
    assert n_blocks % SLAB == 0

    idx = doy.reshape(NW, n_blocks, ROWS).astype(jnp.int32)
    out = _build_kernel(n_rows, d_model, n_blocks)(idx, pe)
    return out.reshape(batch, hist, d_model)


# speedup vs baseline: 18.1968x; 1.0022x over previous
import functools

import jax
import jax.numpy as jnp
from jax import lax
from jax.experimental import pallas as pl
from jax.experimental.pallas import tpu as pltpu
from jax.experimental.pallas import tpu_sc as plsc

NUM_CORES = 2
NUM_SUBCORES = 16
NW = NUM_CORES * NUM_SUBCORES

ROWS = 128
SLAB = 80
NBUF = 4


def _build_kernel(n_rows, d_model, n_blocks):
    mesh = plsc.VectorSubcoreMesh(core_axis_name="c", subcore_axis_name="s")
    n_slabs = n_blocks // SLAB

    @functools.partial(
        pl.kernel,
        out_type=jax.ShapeDtypeStruct((NW, n_blocks, ROWS, d_model), jnp.float32),
        mesh=mesh,
        scratch_types=[
            pltpu.VMEM_SHARED((n_rows, d_model), jnp.float32),
            [pltpu.VMEM((SLAB, ROWS), jnp.int32) for _ in range(2)],
            [pltpu.VMEM((ROWS, d_model), jnp.float32) for _ in range(NBUF)],
            pltpu.SemaphoreType.DMA,
            [pltpu.SemaphoreType.DMA for _ in range(NBUF)],
            pltpu.SemaphoreType.DMA,
        ],
    )
    def gather_kernel(idx_hbm, pe_hbm, out_hbm, table_sp, idx_bufs, rows,
                      gsem, wsems, isem):
        c = lax.axis_index("c")
        s = lax.axis_index("s")
        wid = c * NUM_SUBCORES + s

        @pl.when(s == 0)
        def _():
            pltpu.sync_copy(pe_hbm, table_sp)
        plsc.subcore_barrier()

        def quad_body(si, idx_v, q):
            for b in range(NBUF):
                j = si * SLAB + NBUF * q + b

                @pl.when(j >= NBUF)
                def _():
                    pltpu.make_async_copy(
                        rows[b], out_hbm.at[wid, j], wsems[b]
                    ).wait()

                pltpu.async_copy(
                    table_sp.at[idx_v.at[NBUF * q + b]], rows[b], gsem
                ).wait()
                pltpu.async_copy(rows[b], out_hbm.at[wid, j], wsems[b])

        def slab_pair_body(sp, _):
            si0 = 2 * sp
            si1 = 2 * sp + 1

            pltpu.async_copy(
                idx_hbm.at[wid, pl.ds(si1 * SLAB, SLAB)], idx_bufs[1], isem
            )
            lax.fori_loop(0, SLAB // NBUF,
                          lambda q, _: (quad_body(si0, idx_bufs[0], q), 0)[1], 0)
            pltpu.make_async_copy(
                idx_hbm.at[wid, pl.ds(si1 * SLAB, SLAB)], idx_bufs[1], isem
            ).wait()

            @pl.when(sp + 1 < n_slabs // 2)
            def _():
                pltpu.async_copy(
                    idx_hbm.at[wid, pl.ds((si0 + 2) * SLAB, SLAB)],
                    idx_bufs[0], isem,
                )
            lax.fori_loop(0, SLAB // NBUF,
                          lambda q, _: (quad_body(si1, idx_bufs[1], q), 0)[1], 0)

            @pl.when(sp + 1 < n_slabs // 2)
            def _():
                pltpu.make_async_copy(
                    idx_hbm.at[wid, pl.ds((si0 + 2) * SLAB, SLAB)],
                    idx_bufs[0], isem,
                ).wait()
            return 0

        pltpu.sync_copy(idx_hbm.at[wid, pl.ds(0, SLAB)], idx_bufs[0])
        lax.fori_loop(0, n_slabs // 2, slab_pair_body, 0)

        for b in range(NBUF):
            pltpu.make_async_copy(
                rows[b], out_hbm.at[wid, n_blocks - 1], wsems[b]
            ).wait()

    return gather_kernel


def kernel(doy, pe):
    batch, hist = doy.shape
    n_rows, d_model = pe.shape
    total = batch * hist
    assert total % (NW * ROWS) == 0
    n_blocks = total // (NW * ROWS)
    assert n_blocks % (2 * SLAB) == 0 and SLAB % NBUF == 0

    idx = doy.reshape(NW, n_blocks, ROWS).astype(jnp.int32)
    out = _build_kernel(n_rows, d_model, n_blocks)(idx, pe)
    return out.reshape(batch, hist, d_model)


# speedup vs baseline: 19.4598x; 1.0694x over previous
import functools

import jax
import jax.numpy as jnp
from jax import lax
from jax.experimental import pallas as pl
from jax.experimental.pallas import tpu as pltpu
from jax.experimental.pallas import tpu_sc as plsc

NUM_CORES = 2
NUM_SUBCORES = 16
NW = NUM_CORES * NUM_SUBCORES

ROWS = 128
SLAB = 80
NBUF = 4


def _build_kernel(n_rows, d_model, n_blocks):
    mesh = plsc.VectorSubcoreMesh(core_axis_name="c", subcore_axis_name="s")
    n_slabs = n_blocks // SLAB

    @functools.partial(
        pl.kernel,
        out_type=jax.ShapeDtypeStruct((NW, n_blocks, ROWS, d_model), jnp.float32),
        mesh=mesh,
        scratch_types=[
            pltpu.VMEM_SHARED((n_rows, d_model), jnp.float32),
            [pltpu.VMEM((SLAB, ROWS), jnp.int32) for _ in range(2)],
            [pltpu.VMEM((ROWS, d_model), jnp.float32) for _ in range(NBUF)],
            [pltpu.SemaphoreType.DMA for _ in range(NBUF)],
            [pltpu.SemaphoreType.DMA for _ in range(NBUF)],
            pltpu.SemaphoreType.DMA,
        ],
    )
    def gather_kernel(idx_hbm, pe_hbm, out_hbm, table_sp, idx_bufs, rows,
                      gsems, wsems, isem):
        c = lax.axis_index("c")
        s = lax.axis_index("s")
        wid = c * NUM_SUBCORES + s

        @pl.when(s == 0)
        def _():
            pltpu.sync_copy(pe_hbm, table_sp)
        plsc.subcore_barrier()

        def quad_body(si, idx_v, q):
            for b in range(NBUF):
                j = si * SLAB + NBUF * q + b

                @pl.when(j >= NBUF)
                def _():
                    pltpu.make_async_copy(
                        rows[b], out_hbm.at[wid, j], wsems[b]
                    ).wait()

                pltpu.async_copy(
                    table_sp.at[idx_v.at[NBUF * q + b]], rows[b], gsems[b]
                )

                pb = (b - 1) % NBUF

                def flush_prev():
                    pltpu.make_async_copy(
                        table_sp.at[idx_v.at[0]], rows[pb], gsems[pb]
                    ).wait()
                    pltpu.async_copy(rows[pb], out_hbm.at[wid, j - 1], wsems[pb])

                if b > 0:
                    flush_prev()
                else:
                    pl.when(q >= 1)(flush_prev)

        def slab_flush(si, idx_v):
            last = NBUF - 1
            pltpu.make_async_copy(
                table_sp.at[idx_v.at[0]], rows[last], gsems[last]
            ).wait()
            pltpu.async_copy(
                rows[last], out_hbm.at[wid, si * SLAB + SLAB - 1], wsems[last]
            )

        def slab_pair_body(sp, _):
            si0 = 2 * sp
            si1 = 2 * sp + 1

            pltpu.async_copy(
                idx_hbm.at[wid, pl.ds(si1 * SLAB, SLAB)], idx_bufs[1], isem
            )
            lax.fori_loop(0, SLAB // NBUF,
                          lambda q, _: (quad_body(si0, idx_bufs[0], q), 0)[1], 0)
            slab_flush(si0, idx_bufs[0])
            pltpu.make_async_copy(
                idx_hbm.at[wid, pl.ds(si1 * SLAB, SLAB)], idx_bufs[1], isem
            ).wait()

            @pl.when(sp + 1 < n_slabs // 2)
            def _():
                pltpu.async_copy(
                    idx_hbm.at[wid, pl.ds((si0 + 2) * SLAB, SLAB)],
                    idx_bufs[0], isem,
                )
            lax.fori_loop(0, SLAB // NBUF,
                          lambda q, _: (quad_body(si1, idx_bufs[1], q), 0)[1], 0)
            slab_flush(si1, idx_bufs[1])

            @pl.when(sp + 1 < n_slabs // 2)
            def _():
                pltpu.make_async_copy(
                    idx_hbm.at[wid, pl.ds((si0 + 2) * SLAB, SLAB)],
                    idx_bufs[0], isem,
                ).wait()
            return 0

        pltpu.sync_copy(idx_hbm.at[wid, pl.ds(0, SLAB)], idx_bufs[0])
        lax.fori_loop(0, n_slabs // 2, slab_pair_body, 0)

        for b in range(NBUF):
            pltpu.make_async_copy(
                rows[b], out_hbm.at[wid, n_blocks - 1], wsems[b]
            ).wait()

    return gather_kernel


def kernel(doy, pe):
    batch, hist = doy.shape
    n_rows, d_model = pe.shape
    total = batch * hist
    assert total % (NW * ROWS) == 0
    n_blocks = total // (NW * ROWS)
    assert n_blocks % (2 * SLAB) == 0 and SLAB % NBUF == 0

    idx = doy.reshape(NW, n_blocks, ROWS).astype(jnp.int32)
    out = _build_kernel(n_rows, d_model, n_blocks)(idx, pe)
    return out.reshape(batch, hist, d_model)


# speedup vs baseline: 19.5027x; 1.0022x over previous
import functools

import jax
import jax.numpy as jnp
from jax import lax
from jax.experimental import pallas as pl
from jax.experimental.pallas import tpu as pltpu
from jax.experimental.pallas import tpu_sc as plsc

NUM_CORES = 2
NUM_SUBCORES = 16
NW = NUM_CORES * NUM_SUBCORES

ROWS = 128
SLAB = 80
NBUF = 5


def _build_kernel(n_rows, d_model, n_blocks):
    mesh = plsc.VectorSubcoreMesh(core_axis_name="c", subcore_axis_name="s")
    n_slabs = n_blocks // SLAB

    @functools.partial(
        pl.kernel,
        out_type=jax.ShapeDtypeStruct((NW, n_blocks, ROWS, d_model), jnp.float32),
        mesh=mesh,
        scratch_types=[
            pltpu.VMEM_SHARED((n_rows, d_model), jnp.float32),
            [pltpu.VMEM((SLAB, ROWS), jnp.int32) for _ in range(2)],
            [pltpu.VMEM((ROWS, d_model), jnp.float32) for _ in range(NBUF)],
            [pltpu.SemaphoreType.DMA for _ in range(NBUF)],
            [pltpu.SemaphoreType.DMA for _ in range(NBUF)],
            pltpu.SemaphoreType.DMA,
        ],
    )
    def gather_kernel(idx_hbm, pe_hbm, out_hbm, table_sp, idx_bufs, rows,
                      gsems, wsems, isem):
        c = lax.axis_index("c")
        s = lax.axis_index("s")
        wid = c * NUM_SUBCORES + s

        @pl.when(s == 0)
        def _():
            pltpu.sync_copy(pe_hbm, table_sp)
        plsc.subcore_barrier()

        def quad_body(si, idx_v, q):
            for b in range(NBUF):
                j = si * SLAB + NBUF * q + b

                @pl.when(j >= NBUF)
                def _():
                    pltpu.make_async_copy(
                        rows[b], out_hbm.at[wid, j], wsems[b]
                    ).wait()

                pltpu.async_copy(
                    table_sp.at[idx_v.at[NBUF * q + b]], rows[b], gsems[b]
                )

                pb = (b - 1) % NBUF

                def flush_prev():
                    pltpu.make_async_copy(
                        table_sp.at[idx_v.at[0]], rows[pb], gsems[pb]
                    ).wait()
                    pltpu.async_copy(rows[pb], out_hbm.at[wid, j - 1], wsems[pb])

                if b > 0:
                    flush_prev()
                else:
                    pl.when(q >= 1)(flush_prev)

        def slab_flush(si, idx_v):
            last = NBUF - 1
            pltpu.make_async_copy(
                table_sp.at[idx_v.at[0]], rows[last], gsems[last]
            ).wait()
            pltpu.async_copy(
                rows[last], out_hbm.at[wid, si * SLAB + SLAB - 1], wsems[last]
            )

        def slab_pair_body(sp, _):
            si0 = 2 * sp
            si1 = 2 * sp + 1

            pltpu.async_copy(
                idx_hbm.at[wid, pl.ds(si1 * SLAB, SLAB)], idx_bufs[1], isem
            )
            lax.fori_loop(0, SLAB // NBUF,
                          lambda q, _: (quad_body(si0, idx_bufs[0], q), 0)[1], 0)
            slab_flush(si0, idx_bufs[0])
            pltpu.make_async_copy(
                idx_hbm.at[wid, pl.ds(si1 * SLAB, SLAB)], idx_bufs[1], isem
            ).wait()

            @pl.when(sp + 1 < n_slabs // 2)
            def _():
                pltpu.async_copy(
                    idx_hbm.at[wid, pl.ds((si0 + 2) * SLAB, SLAB)],
                    idx_bufs[0], isem,
                )
            lax.fori_loop(0, SLAB // NBUF,
                          lambda q, _: (quad_body(si1, idx_bufs[1], q), 0)[1], 0)
            slab_flush(si1, idx_bufs[1])

            @pl.when(sp + 1 < n_slabs // 2)
            def _():
                pltpu.make_async_copy(
                    idx_hbm.at[wid, pl.ds((si0 + 2) * SLAB, SLAB)],
                    idx_bufs[0], isem,
                ).wait()
            return 0

        pltpu.sync_copy(idx_hbm.at[wid, pl.ds(0, SLAB)], idx_bufs[0])
        lax.fori_loop(0, n_slabs // 2, slab_pair_body, 0)

        for b in range(NBUF):
            pltpu.make_async_copy(
                rows[b], out_hbm.at[wid, n_blocks - 1], wsems[b]
            ).wait()

    return gather_kernel


def kernel(doy, pe):
    batch, hist = doy.shape
    n_rows, d_model = pe.shape
    total = batch * hist
    assert total % (NW * ROWS) == 0
    n_blocks = total // (NW * ROWS)
    assert n_blocks % (2 * SLAB) == 0 and SLAB % NBUF == 0

    idx = doy.reshape(NW, n_blocks, ROWS).astype(jnp.int32)
    out = _build_kernel(n_rows, d_model, n_blocks)(idx, pe)
    return out.reshape(batch, hist, d_model)
